# Initial kernel scaffold; baseline (speedup 1.0000x reference)
import jax
import jax.numpy as jnp
from jax.experimental import pallas as pl


def kernel(go_inputs, relation_ids, go_table, rel_table):
    raise NotImplementedError("write your pallas kernel here")


# speedup vs baseline: 1.5333x; 1.5333x over previous
import functools

import jax
import jax.numpy as jnp
from jax import lax
from jax.experimental import pallas as pl
from jax.experimental.pallas import tpu as pltpu
from jax.experimental.pallas import tpu_sc as plsc

_VOCAB = 30522
_D = 128
_B = 16384
_CH = 128


@functools.lru_cache(maxsize=1)
def _build():
    info = plsc.get_sparse_core_info()
    nw = info.num_cores * info.num_subcores
    b_per_w = _B // nw
    nch = b_per_w // _CH
    mesh = plsc.VectorSubcoreMesh(core_axis_name="c", subcore_axis_name="s")
    out_sds = jax.ShapeDtypeStruct((_B, _D), jnp.float32)

    @functools.partial(
        pl.kernel,
        mesh=mesh,
        out_type=[out_sds, out_sds],
        scratch_types=[
            pltpu.VMEM((nch, _CH), jnp.int32),
            pltpu.VMEM((b_per_w, _D), jnp.float32),
            pltpu.SemaphoreType.DMA,
        ],
    )
    def sc_gather2(go_idx, rel_idx, go_tab, rel_tab, go_out, rel_out,
                   idx_v, rows_v, sem):
        wid = lax.axis_index("s") * info.num_cores + lax.axis_index("c")
        base = wid * nch

        def one_table(idx_hbm, tab_hbm, out_hbm):
            pltpu.sync_copy(idx_hbm.at[pl.ds(base, nch)], idx_v)
            cps = [
                pltpu.async_copy(tab_hbm.at[idx_v.at[j]],
                                 rows_v.at[pl.ds(j * _CH, _CH)], sem)
                for j in range(nch)
            ]
            for cp in cps:
                cp.wait()
            pltpu.sync_copy(rows_v, out_hbm.at[pl.ds(wid * b_per_w, b_per_w)])

        one_table(go_idx, go_tab, go_out)
        one_table(rel_idx, rel_tab, rel_out)

    return sc_gather2


def kernel(go_inputs, relation_ids, go_table, rel_table):
    k = _build()
    go_idx = go_inputs.astype(jnp.int32).reshape(_B // _CH, _CH)
    rel_idx = relation_ids.astype(jnp.int32).reshape(_B // _CH, _CH)
    entity_embed, relation_embed = k(go_idx, rel_idx, go_table, rel_table)
    return (entity_embed, relation_embed)
